# baseline (speedup 1.0000x reference)
import jax
import jax.numpy as jnp
from jax.experimental import pallas as pl
from jax.experimental.pallas import tpu as pltpu


def _ru(v, m):
    return ((v + m - 1) // m) * m


def _pool_sel(y, m_img, sel_ref, bias_ref, bsz):
    l_out = y.shape[1] // 2
    v_rows = sel_ref.shape[1]
    hm = jnp.maximum(y[:, :l_out], y[:, l_out:])
    vm = jnp.maximum(hm[:-1, :], hm[1:, :])
    vml = jnp.concatenate(
        [vm[b * m_img:b * m_img + v_rows, :] for b in range(bsz)], axis=1)
    z = jnp.dot(sel_ref[...], vml, preferred_element_type=jnp.float32)
    return jnp.maximum(z + bias_ref[...], 0.0)


def _tap_stack(a, k, h_rows, l_img, bsz):
    taps = [a[di:di + h_rows, :] for di in range(k)]
    return jnp.concatenate(
        [jnp.concatenate([t[:, b * l_img:(b + 1) * l_img] for t in taps],
                         axis=1)
         for b in range(bsz)], axis=0)


def _build_x1(x_ref, b0, bsz, k1):
    h = x_ref.shape[1] - (k1 - 1)
    return jnp.concatenate(
        [jnp.concatenate([x_ref[b, di:di + h, :] for di in range(k1)],
                         axis=1)
         for b in range(b0, b0 + bsz)], axis=0)


def _conv_stage(z, bsz, bs_ref, sel_ref, bias_ref):
    bf = jnp.bfloat16
    h_in = z.shape[0]
    l_in = z.shape[1] // bsz
    k = bs_ref.shape[0] // l_in
    a = jnp.concatenate(
        [z.astype(bf), jnp.zeros((k - 1, bsz * l_in), bf)], axis=0)
    xk = _tap_stack(a, k, h_in, l_in, bsz)
    y = jnp.dot(xk, bs_ref[...],
                preferred_element_type=jnp.float32).astype(bf)
    return _pool_sel(y, h_in, sel_ref, bias_ref, bsz)


def _fused_kernel(x_ref, bs1_ref, sel1_ref, bias1_ref,
                  bs2_ref, sel2_ref, bias2_ref,
                  bs3_ref, sel3_ref, bias3_ref,
                  fcc_ref, fcb_ref, o_ref):
    bsz = x_ref.shape[0]
    half = bsz // 2 if bsz % 2 == 0 and bsz > 1 else bsz
    chunks = [(b0, half) for b0 in range(0, bsz, half)]
    l1p = x_ref.shape[2]
    k1 = bs1_ref.shape[0] // l1p
    h = x_ref.shape[1] - (k1 - 1)
    bsc1 = bias1_ref.shape[1] // bsz
    bsc2 = bias2_ref.shape[1] // bsz
    bsc3 = bias3_ref.shape[1] // bsz
    x1s = [_build_x1(x_ref, b0, cs, k1) for (b0, cs) in chunks]
    y1s = [jnp.dot(x1, bs1_ref[...],
                   preferred_element_type=jnp.float32).astype(jnp.bfloat16)
           for x1 in x1s]
    z1s = [_pool_sel(y1, h, sel1_ref, bias1_ref[:, :cs * bsc1], cs)
           for y1, (_, cs) in zip(y1s, chunks)]
    z2s = [_conv_stage(z1, cs, bs2_ref, sel2_ref, bias2_ref[:, :cs * bsc2])
           for z1, (_, cs) in zip(z1s, chunks)]
    z3s = [_conv_stage(z2, cs, bs3_ref, sel3_ref, bias3_ref[:, :cs * bsc3])
           for z2, (_, cs) in zip(z2s, chunks)]
    z3 = z3s[0] if len(z3s) == 1 else jnp.concatenate(z3s, axis=1)

    n_h = fcc_ref.shape[0]
    l4 = z3.shape[1] // bsz
    zb = z3.astype(jnp.bfloat16)
    acc = None
    for ho in range(n_h):
        rows = jnp.concatenate(
            [zb[ho:ho + 1, b * l4:(b + 1) * l4] for b in range(bsz)], axis=0)
        t = jnp.dot(rows, fcc_ref[ho], preferred_element_type=jnp.float32)
        acc = t if acc is None else acc + t
    o_ref[0] = acc + fcb_ref[...]


def kernel(x, s1_band, s1_sel, s1_bias, s2_band, s2_sel, s2_bias,
           s3_band, s3_sel, s3_bias, fcc, fcb):
    n, c, h, w = x.shape
    k1 = s1_band.shape[0]
    bf = jnp.bfloat16
    bsz = next(b for b in (32, 16, 8, 4, 2, 1) if n % b == 0)

    l1p = _ru(w * c, 128)
    xs = jnp.concatenate([x[:, ci] for ci in range(c)], axis=2).astype(bf)
    xs = jnp.pad(xs, ((0, 0), (0, k1 - 1), (0, l1p - w * c)))
    b1p = (s1_band.reshape(k1, w, c, -1).transpose(0, 2, 1, 3)
           .reshape(k1, w * c, -1))
    bs1 = jnp.pad(b1p, ((0, 0), (0, l1p - w * c), (0, 0)))
    bs1 = bs1.reshape(k1 * l1p, -1).astype(bf)
    bs2 = s2_band.reshape(-1, s2_band.shape[2]).astype(bf)
    bs3 = s3_band.reshape(-1, s3_band.shape[2]).astype(bf)

    n_out = fcb.shape[1]
    out = pl.pallas_call(
        _fused_kernel,
        out_shape=jax.ShapeDtypeStruct((n // bsz, bsz, n_out), jnp.float32),
        grid=(n // bsz,),
        in_specs=[
            pl.BlockSpec((bsz, h + k1 - 1, l1p), lambda i: (i, 0, 0)),
            pl.BlockSpec(bs1.shape, lambda i: (0, 0)),
            pl.BlockSpec(s1_sel.shape, lambda i: (0, 0)),
            pl.BlockSpec((1, bsz * s1_bias.shape[1]), lambda i: (0, 0)),
            pl.BlockSpec(bs2.shape, lambda i: (0, 0)),
            pl.BlockSpec(s2_sel.shape, lambda i: (0, 0)),
            pl.BlockSpec((1, bsz * s2_bias.shape[1]), lambda i: (0, 0)),
            pl.BlockSpec(bs3.shape, lambda i: (0, 0)),
            pl.BlockSpec(s3_sel.shape, lambda i: (0, 0)),
            pl.BlockSpec((1, bsz * s3_bias.shape[1]), lambda i: (0, 0)),
            pl.BlockSpec(fcc.shape, lambda i: (0, 0, 0)),
            pl.BlockSpec(fcb.shape, lambda i: (0, 0)),
        ],
        out_specs=pl.BlockSpec((1, bsz, n_out), lambda i: (i, 0, 0)),
        compiler_params=pltpu.CompilerParams(
            dimension_semantics=("parallel",)),
    )(xs, bs1, s1_sel.astype(bf), jnp.tile(s1_bias, (1, bsz)),
      bs2, s2_sel.astype(bf), jnp.tile(s2_bias, (1, bsz)),
      bs3, s3_sel.astype(bf), jnp.tile(s3_bias, (1, bsz)),
      fcc.astype(bf), fcb)
    return out.reshape(n, n_out)


# speedup vs baseline: 1.1513x; 1.1513x over previous
import jax
import jax.numpy as jnp
from jax.experimental import pallas as pl
from jax.experimental.pallas import tpu as pltpu


def _ru(v, m):
    return ((v + m - 1) // m) * m


def _pool_sel(y, m_img, sel_ref, bias_ref, bsz):
    l_out = y.shape[1] // 2
    v_rows = sel_ref.shape[1]
    hm = jnp.maximum(y[:, :l_out], y[:, l_out:])
    vm = jnp.maximum(hm[:-1, :], hm[1:, :])
    vml = jnp.concatenate(
        [vm[b * m_img:b * m_img + v_rows, :] for b in range(bsz)], axis=1)
    z = jnp.dot(sel_ref[...], vml, preferred_element_type=jnp.float32)
    return jnp.maximum(z + bias_ref[...], 0.0)


def _tap_stack(a, k, h_rows, l_img, bsz):
    taps = [a[di:di + h_rows, :] for di in range(k)]
    return jnp.concatenate(
        [jnp.concatenate([t[:, b * l_img:(b + 1) * l_img] for t in taps],
                         axis=1)
         for b in range(bsz)], axis=0)


def _build_x1(x_ref, b0, bsz, k1):
    bf = jnp.bfloat16
    n_c, h, w = x_ref.shape[1], x_ref.shape[2], x_ref.shape[3]
    l1p = _ru(n_c * w, 128)
    slabs = []
    for b in range(b0, b0 + bsz):
        s = jnp.concatenate(
            [x_ref[b, ci] for ci in range(n_c)]
            + ([jnp.zeros((h, l1p - n_c * w), bf)] if l1p > n_c * w else []),
            axis=1)
        slabs.append(jnp.concatenate(
            [s, jnp.zeros((k1 - 1, l1p), bf)], axis=0))
    return jnp.concatenate(
        [jnp.concatenate([sl[di:di + h, :] for di in range(k1)], axis=1)
         for sl in slabs], axis=0)


def _conv_stage(z, bsz, bs_ref, sel_ref, bias_ref):
    bf = jnp.bfloat16
    h_in = z.shape[0]
    l_in = z.shape[1] // bsz
    k = bs_ref.shape[0] // l_in
    a = jnp.concatenate(
        [z.astype(bf), jnp.zeros((k - 1, bsz * l_in), bf)], axis=0)
    xk = _tap_stack(a, k, h_in, l_in, bsz)
    y = jnp.dot(xk, bs_ref[...],
                preferred_element_type=jnp.float32).astype(bf)
    return _pool_sel(y, h_in, sel_ref, bias_ref, bsz)


def _fused_kernel(x_ref, bs1_ref, sel1_ref, bias1_ref,
                  bs2_ref, sel2_ref, bias2_ref,
                  bs3_ref, sel3_ref, bias3_ref,
                  fcc_ref, fcb_ref, o_ref):
    bsz = x_ref.shape[0]
    half = bsz // 2 if bsz % 2 == 0 and bsz > 1 else bsz
    chunks = [(b0, half) for b0 in range(0, bsz, half)]
    l1p = _ru(x_ref.shape[1] * x_ref.shape[3], 128)
    k1 = bs1_ref.shape[0] // l1p
    h = x_ref.shape[2]
    bsc1 = bias1_ref.shape[1] // bsz
    bsc2 = bias2_ref.shape[1] // bsz
    bsc3 = bias3_ref.shape[1] // bsz
    x1s = [_build_x1(x_ref, b0, cs, k1) for (b0, cs) in chunks]
    y1s = [jnp.dot(x1, bs1_ref[...],
                   preferred_element_type=jnp.float32).astype(jnp.bfloat16)
           for x1 in x1s]
    z1s = [_pool_sel(y1, h, sel1_ref, bias1_ref[:, :cs * bsc1], cs)
           for y1, (_, cs) in zip(y1s, chunks)]
    z2s = [_conv_stage(z1, cs, bs2_ref, sel2_ref, bias2_ref[:, :cs * bsc2])
           for z1, (_, cs) in zip(z1s, chunks)]
    z3s = [_conv_stage(z2, cs, bs3_ref, sel3_ref, bias3_ref[:, :cs * bsc3])
           for z2, (_, cs) in zip(z2s, chunks)]
    z3 = z3s[0] if len(z3s) == 1 else jnp.concatenate(z3s, axis=1)

    n_h = fcc_ref.shape[0]
    l4 = z3.shape[1] // bsz
    zb = z3.astype(jnp.bfloat16)
    acc = None
    for ho in range(n_h):
        rows = jnp.concatenate(
            [zb[ho:ho + 1, b * l4:(b + 1) * l4] for b in range(bsz)], axis=0)
        t = jnp.dot(rows, fcc_ref[ho], preferred_element_type=jnp.float32)
        acc = t if acc is None else acc + t
    o_ref[0] = acc + fcb_ref[...]


def kernel(x, s1_band, s1_sel, s1_bias, s2_band, s2_sel, s2_bias,
           s3_band, s3_sel, s3_bias, fcc, fcb):
    n, c, h, w = x.shape
    k1 = s1_band.shape[0]
    bf = jnp.bfloat16
    bsz = next(b for b in (32, 16, 8, 4, 2, 1) if n % b == 0)

    l1p = _ru(w * c, 128)
    b1p = (s1_band.reshape(k1, w, c, -1).transpose(0, 2, 1, 3)
           .reshape(k1, w * c, -1))
    bs1 = jnp.pad(b1p, ((0, 0), (0, l1p - w * c), (0, 0)))
    bs1 = bs1.reshape(k1 * l1p, -1).astype(bf)
    bs2 = s2_band.reshape(-1, s2_band.shape[2]).astype(bf)
    bs3 = s3_band.reshape(-1, s3_band.shape[2]).astype(bf)

    n_out = fcb.shape[1]
    out = pl.pallas_call(
        _fused_kernel,
        out_shape=jax.ShapeDtypeStruct((n // bsz, bsz, n_out), jnp.float32),
        grid=(n // bsz,),
        in_specs=[
            pl.BlockSpec((bsz, c, h, w), lambda i: (i, 0, 0, 0)),
            pl.BlockSpec(bs1.shape, lambda i: (0, 0)),
            pl.BlockSpec(s1_sel.shape, lambda i: (0, 0)),
            pl.BlockSpec((1, bsz * s1_bias.shape[1]), lambda i: (0, 0)),
            pl.BlockSpec(bs2.shape, lambda i: (0, 0)),
            pl.BlockSpec(s2_sel.shape, lambda i: (0, 0)),
            pl.BlockSpec((1, bsz * s2_bias.shape[1]), lambda i: (0, 0)),
            pl.BlockSpec(bs3.shape, lambda i: (0, 0)),
            pl.BlockSpec(s3_sel.shape, lambda i: (0, 0)),
            pl.BlockSpec((1, bsz * s3_bias.shape[1]), lambda i: (0, 0)),
            pl.BlockSpec(fcc.shape, lambda i: (0, 0, 0)),
            pl.BlockSpec(fcb.shape, lambda i: (0, 0)),
        ],
        out_specs=pl.BlockSpec((1, bsz, n_out), lambda i: (i, 0, 0)),
        compiler_params=pltpu.CompilerParams(
            dimension_semantics=("parallel",)),
    )(x.astype(bf), bs1, s1_sel.astype(bf), jnp.tile(s1_bias, (1, bsz)),
      bs2, s2_sel.astype(bf), jnp.tile(s2_bias, (1, bsz)),
      bs3, s3_sel.astype(bf), jnp.tile(s3_bias, (1, bsz)),
      fcc.astype(bf), fcb)
    return out.reshape(n, n_out)


# speedup vs baseline: 1.1601x; 1.0076x over previous
import jax
import jax.numpy as jnp
from jax.experimental import pallas as pl
from jax.experimental.pallas import tpu as pltpu


def _ru(v, m):
    return ((v + m - 1) // m) * m


def _pool_sel(y, m_img, sel_ref, bias_ref, bsz):
    l_out = y.shape[1] // 2
    v_rows = sel_ref.shape[1]
    hm = jnp.maximum(y[:, :l_out], y[:, l_out:])
    vm = jnp.maximum(hm[:-1, :], hm[1:, :])
    vml = jnp.concatenate(
        [vm[b * m_img:b * m_img + v_rows, :] for b in range(bsz)], axis=1)
    z = jnp.dot(sel_ref[...], vml, preferred_element_type=jnp.float32)
    return jnp.maximum(z + bias_ref[...], 0.0)


def _tap_stack(a, k, h_rows, l_img, bsz):
    taps = [a[di:di + h_rows, :] for di in range(k)]
    return jnp.concatenate(
        [jnp.concatenate([t[:, b * l_img:(b + 1) * l_img] for t in taps],
                         axis=1)
         for b in range(bsz)], axis=0)


def _build_x1(x_ref, b0, bsz, k1):
    bf = jnp.bfloat16
    n_c, h, w = x_ref.shape[1], x_ref.shape[2], x_ref.shape[3]
    l1p = _ru(n_c * w, 128)
    slabs = []
    for b in range(b0, b0 + bsz):
        s = jnp.concatenate(
            [x_ref[b, ci].astype(bf) for ci in range(n_c)]
            + ([jnp.zeros((h, l1p - n_c * w), bf)] if l1p > n_c * w else []),
            axis=1)
        slabs.append(jnp.concatenate(
            [s, jnp.zeros((k1 - 1, l1p), bf)], axis=0))
    return jnp.concatenate(
        [jnp.concatenate([sl[di:di + h, :] for di in range(k1)], axis=1)
         for sl in slabs], axis=0)


def _conv_stage(z, bsz, bs_ref, sel_ref, bias_ref):
    bf = jnp.bfloat16
    h_in = z.shape[0]
    l_in = z.shape[1] // bsz
    k = bs_ref.shape[0] // l_in
    a = jnp.concatenate(
        [z.astype(bf), jnp.zeros((k - 1, bsz * l_in), bf)], axis=0)
    xk = _tap_stack(a, k, h_in, l_in, bsz)
    y = jnp.dot(xk, bs_ref[...],
                preferred_element_type=jnp.float32).astype(bf)
    return _pool_sel(y, h_in, sel_ref, bias_ref, bsz)


def _fused_kernel(x_ref, bs1_ref, sel1_ref, bias1_ref,
                  bs2_ref, sel2_ref, bias2_ref,
                  bs3_ref, sel3_ref, bias3_ref,
                  fcc_ref, fcb_ref, o_ref):
    bsz = x_ref.shape[0]
    half = bsz // 2 if bsz % 2 == 0 and bsz > 1 else bsz
    chunks = [(b0, half) for b0 in range(0, bsz, half)]
    l1p = _ru(x_ref.shape[1] * x_ref.shape[3], 128)
    k1 = bs1_ref.shape[0] // l1p
    h = x_ref.shape[2]
    bsc1 = bias1_ref.shape[1] // bsz
    bsc2 = bias2_ref.shape[1] // bsz
    bsc3 = bias3_ref.shape[1] // bsz
    x1s = [_build_x1(x_ref, b0, cs, k1) for (b0, cs) in chunks]
    y1s = [jnp.dot(x1, bs1_ref[...],
                   preferred_element_type=jnp.float32).astype(jnp.bfloat16)
           for x1 in x1s]
    z1s = [_pool_sel(y1, h, sel1_ref, bias1_ref[:, :cs * bsc1], cs)
           for y1, (_, cs) in zip(y1s, chunks)]
    z2s = [_conv_stage(z1, cs, bs2_ref, sel2_ref, bias2_ref[:, :cs * bsc2])
           for z1, (_, cs) in zip(z1s, chunks)]
    z3s = [_conv_stage(z2, cs, bs3_ref, sel3_ref, bias3_ref[:, :cs * bsc3])
           for z2, (_, cs) in zip(z2s, chunks)]
    z3 = z3s[0] if len(z3s) == 1 else jnp.concatenate(z3s, axis=1)

    n_h = fcc_ref.shape[0]
    l4 = z3.shape[1] // bsz
    zb = z3.astype(jnp.bfloat16)
    acc = None
    for ho in range(n_h):
        rows = jnp.concatenate(
            [zb[ho:ho + 1, b * l4:(b + 1) * l4] for b in range(bsz)], axis=0)
        t = jnp.dot(rows, fcc_ref[ho], preferred_element_type=jnp.float32)
        acc = t if acc is None else acc + t
    o_ref[0] = acc + fcb_ref[...]


def kernel(x, s1_band, s1_sel, s1_bias, s2_band, s2_sel, s2_bias,
           s3_band, s3_sel, s3_bias, fcc, fcb):
    n, c, h, w = x.shape
    k1 = s1_band.shape[0]
    bf = jnp.bfloat16
    bsz = next(b for b in (32, 16, 8, 4, 2, 1) if n % b == 0)

    l1p = _ru(w * c, 128)
    b1p = (s1_band.reshape(k1, w, c, -1).transpose(0, 2, 1, 3)
           .reshape(k1, w * c, -1))
    bs1 = jnp.pad(b1p, ((0, 0), (0, l1p - w * c), (0, 0)))
    bs1 = bs1.reshape(k1 * l1p, -1).astype(bf)
    bs2 = s2_band.reshape(-1, s2_band.shape[2]).astype(bf)
    bs3 = s3_band.reshape(-1, s3_band.shape[2]).astype(bf)

    n_out = fcb.shape[1]
    out = pl.pallas_call(
        _fused_kernel,
        out_shape=jax.ShapeDtypeStruct((n // bsz, bsz, n_out), jnp.float32),
        grid=(n // bsz,),
        in_specs=[
            pl.BlockSpec((bsz, c, h, w), lambda i: (i, 0, 0, 0)),
            pl.BlockSpec(bs1.shape, lambda i: (0, 0)),
            pl.BlockSpec(s1_sel.shape, lambda i: (0, 0)),
            pl.BlockSpec((1, bsz * s1_bias.shape[1]), lambda i: (0, 0)),
            pl.BlockSpec(bs2.shape, lambda i: (0, 0)),
            pl.BlockSpec(s2_sel.shape, lambda i: (0, 0)),
            pl.BlockSpec((1, bsz * s2_bias.shape[1]), lambda i: (0, 0)),
            pl.BlockSpec(bs3.shape, lambda i: (0, 0)),
            pl.BlockSpec(s3_sel.shape, lambda i: (0, 0)),
            pl.BlockSpec((1, bsz * s3_bias.shape[1]), lambda i: (0, 0)),
            pl.BlockSpec(fcc.shape, lambda i: (0, 0, 0)),
            pl.BlockSpec(fcb.shape, lambda i: (0, 0)),
        ],
        out_specs=pl.BlockSpec((1, bsz, n_out), lambda i: (i, 0, 0)),
        compiler_params=pltpu.CompilerParams(
            dimension_semantics=("parallel",)),
    )(x, bs1, s1_sel.astype(bf), jnp.tile(s1_bias, (1, bsz)),
      bs2, s2_sel.astype(bf), jnp.tile(s2_bias, (1, bsz)),
      bs3, s3_sel.astype(bf), jnp.tile(s3_bias, (1, bsz)),
      fcc.astype(bf), fcb)
    return out.reshape(n, n_out)


# speedup vs baseline: 1.2156x; 1.0479x over previous
import jax
import jax.numpy as jnp
from jax.experimental import pallas as pl
from jax.experimental.pallas import tpu as pltpu


def _pool_sel(y, m_img, sel_ref, bias_ref, bsz):
    l_out = y.shape[1] // 2
    v_rows = sel_ref.shape[1]
    hm = jnp.maximum(y[:, :l_out], y[:, l_out:])
    vm = jnp.maximum(hm[:-1, :], hm[1:, :])
    vml = jnp.concatenate(
        [vm[b * m_img:b * m_img + v_rows, :] for b in range(bsz)], axis=1)
    z = jnp.dot(sel_ref[...], vml, preferred_element_type=jnp.float32)
    return jnp.maximum(z + bias_ref[...], 0.0)


def _tap_stack(a, k, h_rows, l_img, bsz):
    taps = [a[di:di + h_rows, :] for di in range(k)]
    return jnp.concatenate(
        [jnp.concatenate([t[:, b * l_img:(b + 1) * l_img] for t in taps],
                         axis=1)
         for b in range(bsz)], axis=0)


def _build_x1(x_ref, b0, bsz, k1):
    bf = jnp.bfloat16
    n_c, h, w = x_ref.shape[1], x_ref.shape[2], x_ref.shape[3]
    l1p = n_c * w
    slabs = []
    for b in range(b0, b0 + bsz):
        s = jnp.concatenate(
            [x_ref[b, ci].astype(bf) for ci in range(n_c)]
            + ([jnp.zeros((h, l1p - n_c * w), bf)] if l1p > n_c * w else []),
            axis=1)
        slabs.append(jnp.concatenate(
            [s, jnp.zeros((k1 - 1, l1p), bf)], axis=0))
    return jnp.concatenate(
        [jnp.concatenate([sl[di:di + h, :] for di in range(k1)], axis=1)
         for sl in slabs], axis=0)


def _conv_stage(z, bsz, bs_ref, sel_ref, bias_ref):
    bf = jnp.bfloat16
    h_in = z.shape[0]
    l_in = z.shape[1] // bsz
    k = bs_ref.shape[0] // l_in
    a = jnp.concatenate(
        [z.astype(bf), jnp.zeros((k - 1, bsz * l_in), bf)], axis=0)
    xk = _tap_stack(a, k, h_in, l_in, bsz)
    y = jnp.dot(xk, bs_ref[...],
                preferred_element_type=jnp.float32).astype(bf)
    return _pool_sel(y, h_in, sel_ref, bias_ref, bsz)


def _fused_kernel(x_ref, bs1_ref, sel1_ref, bias1_ref,
                  bs2_ref, sel2_ref, bias2_ref,
                  bs3_ref, sel3_ref, bias3_ref,
                  fcc_ref, fcb_ref, o_ref):
    bsz = x_ref.shape[0]
    half = bsz // 2 if bsz % 2 == 0 and bsz > 1 else bsz
    chunks = [(b0, half) for b0 in range(0, bsz, half)]
    l1p = x_ref.shape[1] * x_ref.shape[3]
    k1 = bs1_ref.shape[0] // l1p
    h = x_ref.shape[2]
    bsc1 = bias1_ref.shape[1] // bsz
    bsc2 = bias2_ref.shape[1] // bsz
    bsc3 = bias3_ref.shape[1] // bsz
    x1s = [_build_x1(x_ref, b0, cs, k1) for (b0, cs) in chunks]
    y1s = [jnp.dot(x1, bs1_ref[...],
                   preferred_element_type=jnp.float32).astype(jnp.bfloat16)
           for x1 in x1s]
    z1s = [_pool_sel(y1, h, sel1_ref, bias1_ref[:, :cs * bsc1], cs)
           for y1, (_, cs) in zip(y1s, chunks)]
    z2s = [_conv_stage(z1, cs, bs2_ref, sel2_ref, bias2_ref[:, :cs * bsc2])
           for z1, (_, cs) in zip(z1s, chunks)]
    z3s = [_conv_stage(z2, cs, bs3_ref, sel3_ref, bias3_ref[:, :cs * bsc3])
           for z2, (_, cs) in zip(z2s, chunks)]

    n_h = fcc_ref.shape[0]
    accs = []
    for z3, (_, cs) in zip(z3s, chunks):
        l4 = z3.shape[1] // cs
        zb = z3.astype(jnp.bfloat16)
        acc = None
        for ho in range(n_h):
            rows = jnp.concatenate(
                [zb[ho:ho + 1, b * l4:(b + 1) * l4] for b in range(cs)],
                axis=0)
            t = jnp.dot(rows, fcc_ref[ho], preferred_element_type=jnp.float32)
            acc = t if acc is None else acc + t
        accs.append(acc + fcb_ref[...])
    o_ref[0] = accs[0] if len(accs) == 1 else jnp.concatenate(accs, axis=0)


def kernel(x, s1_band, s1_sel, s1_bias, s2_band, s2_sel, s2_bias,
           s3_band, s3_sel, s3_bias, fcc, fcb):
    n, c, h, w = x.shape
    k1 = s1_band.shape[0]
    bf = jnp.bfloat16
    bsz = next(b for b in (32, 16, 8, 4, 2, 1) if n % b == 0)

    bs1 = (s1_band.reshape(k1, w, c, -1).transpose(0, 2, 1, 3)
           .reshape(k1 * w * c, -1).astype(bf))
    bs2 = s2_band.reshape(-1, s2_band.shape[2]).astype(bf)
    bs3 = s3_band.reshape(-1, s3_band.shape[2]).astype(bf)

    n_out = fcb.shape[1]
    out = pl.pallas_call(
        _fused_kernel,
        out_shape=jax.ShapeDtypeStruct((n // bsz, bsz, n_out), jnp.float32),
        grid=(n // bsz,),
        in_specs=[
            pl.BlockSpec((bsz, c, h, w), lambda i: (i, 0, 0, 0)),
            pl.BlockSpec(bs1.shape, lambda i: (0, 0)),
            pl.BlockSpec(s1_sel.shape, lambda i: (0, 0)),
            pl.BlockSpec((1, bsz * s1_bias.shape[1]), lambda i: (0, 0)),
            pl.BlockSpec(bs2.shape, lambda i: (0, 0)),
            pl.BlockSpec(s2_sel.shape, lambda i: (0, 0)),
            pl.BlockSpec((1, bsz * s2_bias.shape[1]), lambda i: (0, 0)),
            pl.BlockSpec(bs3.shape, lambda i: (0, 0)),
            pl.BlockSpec(s3_sel.shape, lambda i: (0, 0)),
            pl.BlockSpec((1, bsz * s3_bias.shape[1]), lambda i: (0, 0)),
            pl.BlockSpec(fcc.shape, lambda i: (0, 0, 0)),
            pl.BlockSpec(fcb.shape, lambda i: (0, 0)),
        ],
        out_specs=pl.BlockSpec((1, bsz, n_out), lambda i: (i, 0, 0)),
        compiler_params=pltpu.CompilerParams(
            dimension_semantics=("parallel",)),
    )(x, bs1, s1_sel.astype(bf), jnp.tile(s1_bias, (1, bsz)),
      bs2, s2_sel.astype(bf), jnp.tile(s2_bias, (1, bsz)),
      bs3, s3_sel.astype(bf), jnp.tile(s3_bias, (1, bsz)),
      fcc.astype(bf), fcb)
    return out.reshape(n, n_out)
